# Initial kernel scaffold; baseline (speedup 1.0000x reference)
import jax
import jax.numpy as jnp
from jax.experimental import pallas as pl


def kernel(x, pos, fps_pos, batch, frac_pos, trans_vec, scale, W1, b1, W2, b2):
    raise NotImplementedError("write your pallas kernel here")


# speedup vs baseline: 3.8540x; 3.8540x over previous
import functools

import jax
import jax.numpy as jnp
from jax.experimental import pallas as pl
from jax.experimental.pallas import tpu as pltpu

_B = 8
_NP = 1024
_C = 128
_K = 64
_R = 0.15
_H1 = 128
_H2 = 128

_TI = 128
_TJ = 128
_NI = _NP // _TI
_NJ = _NP // _TJ
_BS_ITERS = 50
_NEG = -1e30


def _prologue_kernel(x_ref, pos_ref, frac_ref, fracT_ref, tv_ref,
                     w1x_ref, w1p_ref, b1_ref,
                     a_ref, c_ref, d2_ref, thr_ref):
    x = x_ref[0]
    pos = pos_ref[0]
    f = frac_ref[0]
    ft = fracT_ref[0]
    tv = tv_ref[0]
    w1x = w1x_ref[...]
    w1p = w1p_ref[...]
    b1 = b1_ref[...]

    pc = (pos[:, 0:1] * w1p[0:1, :]
          + pos[:, 1:2] * w1p[1:2, :]
          + pos[:, 2:3] * w1p[2:3, :])
    a = jnp.dot(x.astype(jnp.bfloat16), w1x_ref[...].astype(jnp.bfloat16),
                preferred_element_type=jnp.float32) + pc + b1
    a_ref[0] = a
    c_ref[0] = -pc

    bf = jnp.bfloat16
    f32 = jnp.float32
    d0 = (f[:, 0:1] - ft[0:1, :]).astype(bf).astype(f32)
    d1 = (f[:, 1:2] - ft[1:2, :]).astype(bf).astype(f32)
    d2c = (f[:, 2:3] - ft[2:3, :]).astype(bf).astype(f32)
    tvb = tv.astype(bf).astype(f32)
    dsq = None
    for l in range(3):
        dv = d0 * tvb[0, l] + d1 * tvb[1, l] + d2c * tvb[2, l]
        dsq = dv * dv if dsq is None else dsq + dv * dv
    d2_ref[0] = dsq

    rowmax = jnp.max(dsq, axis=1, keepdims=True)
    lo = jnp.zeros_like(rowmax)
    hi = rowmax

    def body(_, carry):
        lo, hi = carry
        mid = 0.5 * (lo + hi)
        cnt = jnp.sum((dsq <= mid).astype(jnp.float32), axis=1,
                      keepdims=True)
        ge = cnt >= float(_K)
        return jnp.where(ge, lo, mid), jnp.where(ge, mid, hi)

    lo, hi = jax.lax.fori_loop(0, _BS_ITERS, body, (lo, hi))
    thr_ref[0] = hi


def _pair_kernel(a_ref, c_ref, d2_ref, thr_ref, r2_ref, w2_ref, b2_ref,
                 out_ref):
    j = pl.program_id(2)
    aj = a_ref[0]
    ci = c_ref[0]
    d23 = d2_ref[0]
    thr3 = thr_ref[0]
    r2 = r2_ref[0]

    h1 = jnp.maximum(aj[None, :, :] + ci[:, None, :], 0.0)
    h1f = h1.reshape(_TI * _TJ, _H1).astype(jnp.bfloat16)
    h2 = jnp.dot(h1f, w2_ref[...].astype(jnp.bfloat16),
                 preferred_element_type=jnp.float32)
    h2 = jnp.maximum(h2 + b2_ref[...], 0.0).reshape(_TI, _TJ, _H2)

    mask = jnp.logical_and(d23 <= thr3, d23 < r2[0, 0])
    h2 = jnp.where(mask, h2, _NEG)
    chunk = jnp.max(h2, axis=1)

    @pl.when(j == 0)
    def _init():
        out_ref[0] = chunk

    @pl.when(j > 0)
    def _acc():
        out_ref[0] = jnp.maximum(out_ref[0], chunk)


def kernel(x, pos, fps_pos, batch, frac_pos, trans_vec, scale, W1, b1, W2,
           b2):
    del fps_pos, batch
    xg = x.reshape(_B, _NP, _C)
    posg = pos.reshape(_B, _NP, 3)
    fracg = frac_pos.reshape(_B, _NP, 3)
    fracT = fracg.transpose(0, 2, 1)
    w1x = W1[:_C]
    w1p = W1[_C:]
    b1r = b1.reshape(1, _H1)
    b2r = b2.reshape(1, _H2)
    r2 = ((_R / scale) ** 2).reshape(_B, 1, 1).astype(jnp.float32)

    a, c, d2, thr = pl.pallas_call(
        _prologue_kernel,
        grid=(_B,),
        in_specs=[
            pl.BlockSpec((1, _NP, _C), lambda b: (b, 0, 0)),
            pl.BlockSpec((1, _NP, 3), lambda b: (b, 0, 0)),
            pl.BlockSpec((1, _NP, 3), lambda b: (b, 0, 0)),
            pl.BlockSpec((1, 3, _NP), lambda b: (b, 0, 0)),
            pl.BlockSpec((1, 3, 3), lambda b: (b, 0, 0)),
            pl.BlockSpec((_C, _H1), lambda b: (0, 0)),
            pl.BlockSpec((3, _H1), lambda b: (0, 0)),
            pl.BlockSpec((1, _H1), lambda b: (0, 0)),
        ],
        out_specs=[
            pl.BlockSpec((1, _NP, _H1), lambda b: (b, 0, 0)),
            pl.BlockSpec((1, _NP, _H1), lambda b: (b, 0, 0)),
            pl.BlockSpec((1, _NP, _NP), lambda b: (b, 0, 0)),
            pl.BlockSpec((1, _NP, 1), lambda b: (b, 0, 0)),
        ],
        out_shape=[
            jax.ShapeDtypeStruct((_B, _NP, _H1), jnp.float32),
            jax.ShapeDtypeStruct((_B, _NP, _H1), jnp.float32),
            jax.ShapeDtypeStruct((_B, _NP, _NP), jnp.float32),
            jax.ShapeDtypeStruct((_B, _NP, 1), jnp.float32),
        ],
    )(xg, posg, fracg, fracT, trans_vec, w1x, w1p, b1r)

    d2r = d2.reshape(_B, _NP, _NP, 1)
    thrr = thr.reshape(_B, _NP, 1, 1)
    out = pl.pallas_call(
        _pair_kernel,
        grid=(_B, _NI, _NJ),
        in_specs=[
            pl.BlockSpec((1, _TJ, _H1), lambda b, i, j: (b, j, 0)),
            pl.BlockSpec((1, _TI, _H1), lambda b, i, j: (b, i, 0)),
            pl.BlockSpec((1, _TI, _TJ, 1), lambda b, i, j: (b, i, j, 0)),
            pl.BlockSpec((1, _TI, 1, 1), lambda b, i, j: (b, i, 0, 0)),
            pl.BlockSpec((1, 1, 1), lambda b, i, j: (b, 0, 0)),
            pl.BlockSpec((_H1, _H2), lambda b, i, j: (0, 0)),
            pl.BlockSpec((1, _H2), lambda b, i, j: (0, 0)),
        ],
        out_specs=pl.BlockSpec((1, _TI, _H2), lambda b, i, j: (b, i, 0)),
        out_shape=jax.ShapeDtypeStruct((_B, _NP, _H2), jnp.float32),
        compiler_params=pltpu.CompilerParams(
            dimension_semantics=("parallel", "parallel", "arbitrary")),
    )(a, c, d2r, thrr, r2, W2, b2r)

    return out.reshape(_B * _NP, _H2)
